# baseline (speedup 1.0000x reference)
import dataclasses
import functools

import jax
import jax.numpy as jnp
from jax import lax
from jax.experimental import pallas as pl
from jax.experimental.pallas import tpu as pltpu
from jax.experimental.pallas import tpu_sc as plsc

N = 10000
F = 128
E = 320000
NC = 2
NS = 16
NW = NC * NS
CH = 128
NCHUNK = 80
HC = NCHUNK // 2
EPW = NCHUNK * CH
ET = NW * EPW
GRP = CH // 16
RSUB = 624
DCH = 640

_mesh = plsc.VectorSubcoreMesh(core_axis_name="c", subcore_axis_name="s")

_sc_params = pltpu.CompilerParams()
if "needs_layout_passes" in pltpu.CompilerParams.__dataclass_fields__:
    _sc_params = dataclasses.replace(_sc_params, needs_layout_passes=False)


@functools.partial(
    pl.kernel,
    out_type=(
        jax.ShapeDtypeStruct((NC, N, F), jnp.float32),
        jax.ShapeDtypeStruct((NC, N), jnp.float32),
    ),
    mesh=_mesh,
    compiler_params=_sc_params,
    scratch_types=[
        pltpu.VMEM((HC * CH,), jnp.int32),
        pltpu.VMEM((HC, CH), jnp.int32),
        pltpu.VMEM((2, CH), jnp.float32),
        pltpu.VMEM((2, CH), jnp.float32),
        pltpu.VMEM((2, CH, F), jnp.float32),
        pltpu.VMEM((2, CH), jnp.float32),
        pltpu.VMEM_SHARED((N, F), jnp.float32),
        pltpu.VMEM_SHARED((N,), jnp.float32),
        pltpu.VMEM_SHARED((N,), jnp.float32),
        pltpu.VMEM_SHARED((N,), jnp.float32),
        pltpu.SemaphoreType.DMA,
        pltpu.SemaphoreType.DMA,
        pltpu.SemaphoreType.DMA,
        pltpu.SemaphoreType.DMA,
        pltpu.SemaphoreType.DMA,
        pltpu.SemaphoreType.DMA,
        pltpu.SemaphoreType.DMA,
        pltpu.SemaphoreType.DMA,
        pltpu.SemaphoreType.DMA,
        pltpu.SemaphoreType.DMA,
    ],
)
def _edge_pass(hp_hbm, el_hbm, er_hbm, src_hbm, dst2_hbm,
               u_hbm, d_hbm,
               src_v, dst2_v, elg_v, erg_v, rows_v, ee_v,
               u_sh, d_sh, el_sh, er_sh,
               rsem0, rsem1, lsem0, lsem1, esem0, esem1, dsem0, dsem1,
               usem0, usem1):
    cid = lax.axis_index("c")
    sid = lax.axis_index("s")
    wid = sid * NC + cid

    zero16 = jnp.zeros((16,), jnp.float32)

    @pl.loop(0, CH)
    def _(r):
        for q in range(GRP):
            rows_v[0, r, pl.ds(q * 16, 16)] = zero16

    for q in range(GRP):
        ee_v[0, pl.ds(q * 16, 16)] = zero16

    @pl.when(sid < NS - 1)
    def _():
        @pl.loop(0, 4)
        def _(j):
            pltpu.sync_copy(rows_v.at[0],
                            u_sh.at[pl.ds(sid * RSUB + j * CH, CH)])
        pltpu.sync_copy(rows_v.at[0, pl.ds(0, 112)],
                        u_sh.at[pl.ds(sid * RSUB + 4 * CH, 112)])

    @pl.when(sid == NS - 1)
    def _():
        @pl.loop(0, 5)
        def _(j):
            pltpu.sync_copy(rows_v.at[0],
                            u_sh.at[pl.ds(15 * RSUB + j * CH, CH)])

    @pl.when(sid < NS - 1)
    def _():
        @pl.loop(0, 5)
        def _(k):
            pltpu.sync_copy(ee_v.at[0],
                            d_sh.at[pl.ds(sid * DCH + k * CH, CH)])

    @pl.when(sid == NS - 1)
    def _():
        @pl.loop(0, 3)
        def _(k):
            pltpu.sync_copy(ee_v.at[0], d_sh.at[pl.ds(9600 + k * CH, CH)])
        pltpu.sync_copy(ee_v.at[0, pl.ds(0, 16)], d_sh.at[pl.ds(9984, 16)])

    @pl.when(sid == 0)
    def _():
        pltpu.sync_copy(el_hbm, el_sh)
        pltpu.sync_copy(er_hbm, er_sh)

    plsc.subcore_barrier()

    iota16 = lax.iota(jnp.int32, 16)
    rsems = (rsem0, rsem1)
    lsems = (lsem0, lsem1)
    esems = (esem0, esem1)
    dsems = (dsem0, dsem1)

    usems = (usem0, usem1)

    def _u_scatter_wait(b):
        pltpu.make_async_copy(rows_v.at[b], u_sh.at[dst2_v.at[0]],
                              usems[b]).wait()

    def _d_scatter_wait(b):
        pltpu.make_async_copy(ee_v.at[b], d_sh.at[dst2_v.at[0]],
                              dsems[b]).wait()

    def _gather(ci, b):
        pltpu.async_copy(
            hp_hbm.at[src_v.at[pl.ds(ci * CH, CH)]], rows_v.at[b], rsems[b])
        pltpu.async_copy(
            el_sh.at[src_v.at[pl.ds(ci * CH, CH)]], elg_v.at[b], lsems[b])
        pltpu.async_copy(er_sh.at[dst2_v.at[ci]], erg_v.at[b], esems[b])

    def _gather_wait(ci, b):
        pltpu.make_async_copy(
            hp_hbm.at[src_v.at[pl.ds(ci * CH, CH)]], rows_v.at[b],
            rsems[b]).wait()
        pltpu.make_async_copy(
            el_sh.at[src_v.at[pl.ds(ci * CH, CH)]], elg_v.at[b],
            lsems[b]).wait()
        pltpu.make_async_copy(er_sh.at[dst2_v.at[ci]], erg_v.at[b],
                              esems[b]).wait()

    def _process(hf, ci, b):
        @pl.loop(0, GRP, unroll=2)
        def _(g):
            e = elg_v[b, pl.ds(g * 16, 16)] + erg_v[b, pl.ds(g * 16, 16)]
            e = jnp.maximum(e, 0.2 * e)
            ee = jnp.exp(e)
            gidx = (wid * EPW + hf * (HC * CH) + ci * CH + g * 16) + iota16
            ee = jnp.where(gidx < E, ee, 0.0)
            ee_v[b, pl.ds(g * 16, 16)] = ee
            for r in range(16):
                scale = ee.at[jnp.full((16,), r, jnp.int32)].get(
                    mode="promise_in_bounds")
                row = g * 16 + r
                for q in range(GRP):
                    sl = pl.ds(q * 16, 16)
                    rows_v[b, row, sl] = rows_v[b, row, sl] * scale

        pltpu.async_copy(rows_v.at[b], u_sh.at[dst2_v.at[ci]], usems[b],
                         add=True)
        pltpu.async_copy(ee_v.at[b], d_sh.at[dst2_v.at[ci]], dsems[b],
                         add=True)

    @pl.loop(0, 2)
    def _(hf):
        pltpu.sync_copy(src_hbm.at[wid, pl.ds(hf * (HC * CH), HC * CH)],
                        src_v)
        pltpu.sync_copy(dst2_hbm.at[wid, pl.ds(hf * HC, HC)], dst2_v)

        _gather(0, 0)

        @pl.loop(0, HC // 2)
        def _(p):
            ca = 2 * p
            _gather_wait(ca, 0)

            @pl.when(p > 0)
            def _():
                _u_scatter_wait(1)

            _gather(ca + 1, 1)

            @pl.when(p > 0)
            def _():
                _d_scatter_wait(0)

            _process(hf, ca, 0)
            _gather_wait(ca + 1, 1)

            @pl.when(p < HC // 2 - 1)
            def _():
                _u_scatter_wait(0)
                _gather(ca + 2, 0)

            @pl.when(p > 0)
            def _():
                _d_scatter_wait(1)

            _process(hf, ca + 1, 1)

        _u_scatter_wait(0)
        _u_scatter_wait(1)
        _d_scatter_wait(0)
        _d_scatter_wait(1)

    plsc.subcore_barrier()

    @pl.when(sid < NS - 1)
    def _():
        @pl.loop(0, 4)
        def _(j):
            r0 = sid * RSUB + j * CH
            pltpu.sync_copy(u_sh.at[pl.ds(r0, CH)],
                            u_hbm.at[cid, pl.ds(r0, CH)])
        r1 = sid * RSUB + 4 * CH
        pltpu.sync_copy(u_sh.at[pl.ds(r1, 112)],
                        u_hbm.at[cid, pl.ds(r1, 112)])

    @pl.when(sid == NS - 1)
    def _():
        @pl.loop(0, 5)
        def _(j):
            r0 = 15 * RSUB + j * CH
            pltpu.sync_copy(u_sh.at[pl.ds(r0, CH)],
                            u_hbm.at[cid, pl.ds(r0, CH)])

    @pl.when(sid == 0)
    def _():
        pltpu.sync_copy(d_sh, d_hbm.at[cid])


def _tc_in_body(x_ref, wlin_ref, blin_ref, w_ref, a2_ref, hp_ref, elr_ref):
    h0 = jnp.dot(x_ref[...], wlin_ref[...], preferred_element_type=jnp.float32)
    h0 = jnp.maximum(h0 + blin_ref[...], 0.0)
    hp = jnp.dot(h0, w_ref[...], preferred_element_type=jnp.float32)
    hp_ref[...] = hp
    elr_ref[...] = jnp.dot(hp, a2_ref[...], preferred_element_type=jnp.float32)


def _tc_mid_body(u_ref, d_ref, b_ref, w_ref, a2_ref, h_ref, hp_ref, elr_ref):
    usum = u_ref[0] + u_ref[1]
    dsum = d_ref[0] + d_ref[1]
    h = usum / (dsum + 1e-9)[:, None] + b_ref[...]
    h_ref[...] = h
    hr = jnp.maximum(h, 0.0)
    hp = jnp.dot(hr, w_ref[...], preferred_element_type=jnp.float32)
    hp_ref[...] = hp
    elr_ref[...] = jnp.dot(hp, a2_ref[...], preferred_element_type=jnp.float32)


def _tc_out_body(u_ref, d_ref, b_ref, h1_ref, h2_ref, wr1_ref, wr2_ref,
                 wr3_ref, br_ref, out_ref):
    dsum = d_ref[0] + d_ref[1]
    h3 = (u_ref[0] + u_ref[1]) / (dsum + 1e-9)[:, None] + b_ref[...]
    acc = (jnp.dot(h1_ref[...], wr1_ref[...], preferred_element_type=jnp.float32)
           + jnp.dot(h2_ref[...], wr2_ref[...], preferred_element_type=jnp.float32)
           + jnp.dot(h3, wr3_ref[...], preferred_element_type=jnp.float32))
    out_ref[...] = jax.nn.sigmoid(acc + br_ref[...])


_f32 = jnp.float32
_tc_in = pl.pallas_call(
    _tc_in_body,
    out_shape=[jax.ShapeDtypeStruct((N, F), _f32),
               jax.ShapeDtypeStruct((N, 2), _f32)],
)
_tc_mid = pl.pallas_call(
    _tc_mid_body,
    out_shape=[jax.ShapeDtypeStruct((N, F), _f32),
               jax.ShapeDtypeStruct((N, F), _f32),
               jax.ShapeDtypeStruct((N, 2), _f32)],
)
_tc_out = pl.pallas_call(
    _tc_out_body,
    out_shape=jax.ShapeDtypeStruct((N, 1), _f32),
)


def kernel(x, edge_index, Wlin, blin, W1, al1, ar1, b1, W2, al2, ar2, b2,
           W3, al3, ar3, b3, Wr, br):
    src = edge_index[0]
    dst = edge_index[1]

    fill = (jnp.arange(ET - E, dtype=jnp.int32) + E) % N
    srcp = jnp.concatenate([src, fill]).reshape(NW, EPW)
    dst2 = jnp.concatenate([dst, fill]).reshape(NW, NCHUNK, CH)

    a2_1 = jnp.stack([al1, ar1], axis=1)
    a2_2 = jnp.stack([al2, ar2], axis=1)
    a2_3 = jnp.stack([al3, ar3], axis=1)

    hp1, elr1 = _tc_in(x, Wlin, blin, W1, a2_1)
    u1, d1 = _edge_pass(hp1, elr1[:, 0], elr1[:, 1], srcp, dst2)

    h1, hp2, elr2 = _tc_mid(u1, d1, b1, W2, a2_2)
    u2, d2 = _edge_pass(hp2, elr2[:, 0], elr2[:, 1], srcp, dst2)

    h2, hp3, elr3 = _tc_mid(u2, d2, b2, W3, a2_3)
    u3, d3 = _edge_pass(hp3, elr3[:, 0], elr3[:, 1], srcp, dst2)

    out = _tc_out(u3, d3, b3, h1, h2, Wr[:F], Wr[F:2 * F], Wr[2 * F:], br)
    return out


# speedup vs baseline: 1.0014x; 1.0014x over previous
import dataclasses
import functools

import jax
import jax.numpy as jnp
from jax import lax
from jax.experimental import pallas as pl
from jax.experimental.pallas import tpu as pltpu
from jax.experimental.pallas import tpu_sc as plsc

N = 10000
F = 128
E = 320000
NC = 2
NS = 16
NW = NC * NS
CH = 128
NCHUNK = 80
HC = NCHUNK // 2
EPW = NCHUNK * CH
ET = NW * EPW
GRP = CH // 16
RSUB = 624
DCH = 640

_mesh = plsc.VectorSubcoreMesh(core_axis_name="c", subcore_axis_name="s")

_sc_params = pltpu.CompilerParams()
if "needs_layout_passes" in pltpu.CompilerParams.__dataclass_fields__:
    _sc_params = dataclasses.replace(_sc_params, needs_layout_passes=False)


@functools.partial(
    pl.kernel,
    out_type=(
        jax.ShapeDtypeStruct((NC, N, F), jnp.float32),
        jax.ShapeDtypeStruct((NC, N), jnp.float32),
    ),
    mesh=_mesh,
    compiler_params=_sc_params,
    scratch_types=[
        pltpu.VMEM((HC * CH,), jnp.int32),
        pltpu.VMEM((HC, CH), jnp.int32),
        pltpu.VMEM((2, CH), jnp.float32),
        pltpu.VMEM((2, CH), jnp.float32),
        pltpu.VMEM((2, CH, F), jnp.float32),
        pltpu.VMEM((2, CH), jnp.float32),
        pltpu.VMEM_SHARED((N, F), jnp.float32),
        pltpu.VMEM_SHARED((N,), jnp.float32),
        pltpu.VMEM_SHARED((N,), jnp.float32),
        pltpu.VMEM_SHARED((N,), jnp.float32),
        pltpu.SemaphoreType.DMA,
        pltpu.SemaphoreType.DMA,
        pltpu.SemaphoreType.DMA,
        pltpu.SemaphoreType.DMA,
        pltpu.SemaphoreType.DMA,
        pltpu.SemaphoreType.DMA,
        pltpu.SemaphoreType.DMA,
        pltpu.SemaphoreType.DMA,
    ],
)
def _edge_pass(hp_hbm, el_hbm, er_hbm, src_hbm, dst2_hbm,
               u_hbm, d_hbm,
               src_v, dst2_v, elg_v, erg_v, rows_v, ee_v,
               u_sh, d_sh, el_sh, er_sh,
               rsem0, rsem1, lsem0, lsem1, esem0, esem1, dsem0, dsem1):
    cid = lax.axis_index("c")
    sid = lax.axis_index("s")
    wid = sid * NC + cid

    zero16 = jnp.zeros((16,), jnp.float32)

    @pl.loop(0, CH)
    def _(r):
        for q in range(GRP):
            rows_v[0, r, pl.ds(q * 16, 16)] = zero16

    for q in range(GRP):
        ee_v[0, pl.ds(q * 16, 16)] = zero16

    @pl.when(sid < NS - 1)
    def _():
        @pl.loop(0, 4)
        def _(j):
            pltpu.sync_copy(rows_v.at[0],
                            u_sh.at[pl.ds(sid * RSUB + j * CH, CH)])
        pltpu.sync_copy(rows_v.at[0, pl.ds(0, 112)],
                        u_sh.at[pl.ds(sid * RSUB + 4 * CH, 112)])

    @pl.when(sid == NS - 1)
    def _():
        @pl.loop(0, 5)
        def _(j):
            pltpu.sync_copy(rows_v.at[0],
                            u_sh.at[pl.ds(15 * RSUB + j * CH, CH)])

    @pl.when(sid < NS - 1)
    def _():
        @pl.loop(0, 5)
        def _(k):
            pltpu.sync_copy(ee_v.at[0],
                            d_sh.at[pl.ds(sid * DCH + k * CH, CH)])

    @pl.when(sid == NS - 1)
    def _():
        @pl.loop(0, 3)
        def _(k):
            pltpu.sync_copy(ee_v.at[0], d_sh.at[pl.ds(9600 + k * CH, CH)])
        pltpu.sync_copy(ee_v.at[0, pl.ds(0, 16)], d_sh.at[pl.ds(9984, 16)])

    @pl.when(sid == 0)
    def _():
        pltpu.sync_copy(el_hbm, el_sh)
        pltpu.sync_copy(er_hbm, er_sh)

    plsc.subcore_barrier()

    iota16 = lax.iota(jnp.int32, 16)
    rsems = (rsem0, rsem1)
    lsems = (lsem0, lsem1)
    esems = (esem0, esem1)
    dsems = (dsem0, dsem1)

    def _d_scatter_wait(b):
        pltpu.make_async_copy(ee_v.at[b], d_sh.at[dst2_v.at[0]],
                              dsems[b]).wait()

    def _gather(ci, b):
        pltpu.async_copy(
            hp_hbm.at[src_v.at[pl.ds(ci * CH, CH)]], rows_v.at[b], rsems[b])
        pltpu.async_copy(
            el_sh.at[src_v.at[pl.ds(ci * CH, CH)]], elg_v.at[b], lsems[b])
        pltpu.async_copy(er_sh.at[dst2_v.at[ci]], erg_v.at[b], esems[b])

    def _gather_wait(ci, b):
        pltpu.make_async_copy(
            hp_hbm.at[src_v.at[pl.ds(ci * CH, CH)]], rows_v.at[b],
            rsems[b]).wait()
        pltpu.make_async_copy(
            el_sh.at[src_v.at[pl.ds(ci * CH, CH)]], elg_v.at[b],
            lsems[b]).wait()
        pltpu.make_async_copy(er_sh.at[dst2_v.at[ci]], erg_v.at[b],
                              esems[b]).wait()

    def _process(hf, ci, b):
        @pl.loop(0, GRP, unroll=2)
        def _(g):
            e = elg_v[b, pl.ds(g * 16, 16)] + erg_v[b, pl.ds(g * 16, 16)]
            e = jnp.maximum(e, 0.2 * e)
            ee = jnp.exp(e)
            gidx = (wid * EPW + hf * (HC * CH) + ci * CH + g * 16) + iota16
            ee = jnp.where(gidx < E, ee, 0.0)
            ee_v[b, pl.ds(g * 16, 16)] = ee
            for r in range(16):
                scale = ee.at[jnp.full((16,), r, jnp.int32)].get(
                    mode="promise_in_bounds")
                row = g * 16 + r
                for q in range(GRP):
                    sl = pl.ds(q * 16, 16)
                    rows_v[b, row, sl] = rows_v[b, row, sl] * scale

        pltpu.sync_copy(rows_v.at[b], u_sh.at[dst2_v.at[ci]], add=True)
        pltpu.async_copy(ee_v.at[b], d_sh.at[dst2_v.at[ci]], dsems[b],
                         add=True)

    @pl.loop(0, 2)
    def _(hf):
        pltpu.sync_copy(src_hbm.at[wid, pl.ds(hf * (HC * CH), HC * CH)],
                        src_v)
        pltpu.sync_copy(dst2_hbm.at[wid, pl.ds(hf * HC, HC)], dst2_v)

        _gather(0, 0)

        @pl.loop(0, HC // 2)
        def _(p):
            ca = 2 * p
            _gather_wait(ca, 0)
            _gather(ca + 1, 1)

            @pl.when(p > 0)
            def _():
                _d_scatter_wait(0)

            _process(hf, ca, 0)
            _gather_wait(ca + 1, 1)

            @pl.when(p < HC // 2 - 1)
            def _():
                _gather(ca + 2, 0)

            @pl.when(p > 0)
            def _():
                _d_scatter_wait(1)

            _process(hf, ca + 1, 1)

        _d_scatter_wait(0)
        _d_scatter_wait(1)

    plsc.subcore_barrier()

    @pl.when(sid < NS - 1)
    def _():
        @pl.loop(0, 4)
        def _(j):
            r0 = sid * RSUB + j * CH
            pltpu.sync_copy(u_sh.at[pl.ds(r0, CH)],
                            u_hbm.at[cid, pl.ds(r0, CH)])
        r1 = sid * RSUB + 4 * CH
        pltpu.sync_copy(u_sh.at[pl.ds(r1, 112)],
                        u_hbm.at[cid, pl.ds(r1, 112)])

    @pl.when(sid == NS - 1)
    def _():
        @pl.loop(0, 5)
        def _(j):
            r0 = 15 * RSUB + j * CH
            pltpu.sync_copy(u_sh.at[pl.ds(r0, CH)],
                            u_hbm.at[cid, pl.ds(r0, CH)])

    @pl.when(sid == 0)
    def _():
        pltpu.sync_copy(d_sh, d_hbm.at[cid])


def _tc_in_body(x_ref, wlin_ref, blin_ref, w_ref, a2_ref, hp_ref, elr_ref):
    h0 = jnp.dot(x_ref[...], wlin_ref[...], preferred_element_type=jnp.float32)
    h0 = jnp.maximum(h0 + blin_ref[...], 0.0)
    hp = jnp.dot(h0, w_ref[...], preferred_element_type=jnp.float32)
    hp_ref[...] = hp
    elr_ref[...] = jnp.dot(hp, a2_ref[...], preferred_element_type=jnp.float32)


def _tc_mid_body(u_ref, d_ref, b_ref, w_ref, a2_ref, h_ref, hp_ref, elr_ref):
    usum = u_ref[0] + u_ref[1]
    dsum = d_ref[0] + d_ref[1]
    h = usum / (dsum + 1e-9)[:, None] + b_ref[...]
    h_ref[...] = h
    hr = jnp.maximum(h, 0.0)
    hp = jnp.dot(hr, w_ref[...], preferred_element_type=jnp.float32)
    hp_ref[...] = hp
    elr_ref[...] = jnp.dot(hp, a2_ref[...], preferred_element_type=jnp.float32)


def _tc_out_body(u_ref, d_ref, b_ref, h1_ref, h2_ref, wr1_ref, wr2_ref,
                 wr3_ref, br_ref, out_ref):
    dsum = d_ref[0] + d_ref[1]
    h3 = (u_ref[0] + u_ref[1]) / (dsum + 1e-9)[:, None] + b_ref[...]
    acc = (jnp.dot(h1_ref[...], wr1_ref[...], preferred_element_type=jnp.float32)
           + jnp.dot(h2_ref[...], wr2_ref[...], preferred_element_type=jnp.float32)
           + jnp.dot(h3, wr3_ref[...], preferred_element_type=jnp.float32))
    out_ref[...] = jax.nn.sigmoid(acc + br_ref[...])


_f32 = jnp.float32
_tc_in = pl.pallas_call(
    _tc_in_body,
    out_shape=[jax.ShapeDtypeStruct((N, F), _f32),
               jax.ShapeDtypeStruct((N, 2), _f32)],
)
_tc_mid = pl.pallas_call(
    _tc_mid_body,
    out_shape=[jax.ShapeDtypeStruct((N, F), _f32),
               jax.ShapeDtypeStruct((N, F), _f32),
               jax.ShapeDtypeStruct((N, 2), _f32)],
)
_tc_out = pl.pallas_call(
    _tc_out_body,
    out_shape=jax.ShapeDtypeStruct((N, 1), _f32),
)


def kernel(x, edge_index, Wlin, blin, W1, al1, ar1, b1, W2, al2, ar2, b2,
           W3, al3, ar3, b3, Wr, br):
    src = edge_index[0]
    dst = edge_index[1]

    fill = (jnp.arange(ET - E, dtype=jnp.int32) + E) % N
    srcp = jnp.concatenate([src, fill]).reshape(NW, EPW)
    dst2 = jnp.concatenate([dst, fill]).reshape(NW, NCHUNK, CH)

    a2_1 = jnp.stack([al1, ar1], axis=1)
    a2_2 = jnp.stack([al2, ar2], axis=1)
    a2_3 = jnp.stack([al3, ar3], axis=1)

    hp1, elr1 = _tc_in(x, Wlin, blin, W1, a2_1)
    u1, d1 = _edge_pass(hp1, elr1[:, 0], elr1[:, 1], srcp, dst2)

    h1, hp2, elr2 = _tc_mid(u1, d1, b1, W2, a2_2)
    u2, d2 = _edge_pass(hp2, elr2[:, 0], elr2[:, 1], srcp, dst2)

    h2, hp3, elr3 = _tc_mid(u2, d2, b2, W3, a2_3)
    u3, d3 = _edge_pass(hp3, elr3[:, 0], elr3[:, 1], srcp, dst2)

    out = _tc_out(u3, d3, b3, h1, h2, Wr[:F], Wr[F:2 * F], Wr[2 * F:], br)
    return out
